# baseline (speedup 1.0000x reference)
import jax
import jax.numpy as jnp
from jax.experimental import pallas as pl

B = 16
D = 64
K = 4
C_BLK = 10240


def _body(codes_ref, cents_ref, out_ref):
    k = pl.program_id(1)
    codes = codes_ref[...]
    ones = jnp.ones((1, D), jnp.float32)
    ck = cents_ref[0]
    sim = jax.lax.dot_general(
        codes, ck, (((1,), (0,)), ((), ())),
        preferred_element_type=jnp.float32)
    n2 = jax.lax.dot_general(
        ones, ck * ck, (((1,), (0,)), ((), ())),
        preferred_element_type=jnp.float32)
    inv = jax.lax.rsqrt(jnp.maximum(n2, 1e-24))
    s = sim * inv

    @pl.when(k == 0)
    def _init():
        out_ref[...] = s

    @pl.when(jnp.logical_and(k > 0, k < K - 1))
    def _acc():
        out_ref[...] = jnp.maximum(out_ref[...], s)

    @pl.when(k == K - 1)
    def _fin():
        out_ref[...] = 1.0 - jnp.maximum(out_ref[...], s)


@jax.jit
def kernel(codes, centroids):
    c = centroids.shape[0]
    cents_t = jnp.transpose(centroids, (1, 2, 0))
    grid = ((c + C_BLK - 1) // C_BLK, K)
    return pl.pallas_call(
        _body,
        grid=grid,
        in_specs=[
            pl.BlockSpec((B, D), lambda i, k: (0, 0)),
            pl.BlockSpec((1, D, C_BLK), lambda i, k: (k, 0, i)),
        ],
        out_specs=pl.BlockSpec((B, C_BLK), lambda i, k: (0, i)),
        out_shape=jax.ShapeDtypeStruct((B, c), jnp.float32),
    )(codes, cents_t)
